# Initial kernel scaffold; baseline (speedup 1.0000x reference)
import jax
import jax.numpy as jnp
from jax.experimental import pallas as pl


def kernel(edge_feat, node_feat, graph_attr, edge_index, params):
    raise NotImplementedError("write your pallas kernel here")


# speedup vs baseline: 2.3074x; 2.3074x over previous
import functools

import jax
import jax.numpy as jnp
from jax import lax
from jax.experimental import pallas as pl
from jax.experimental.pallas import tpu as pltpu
from jax.experimental.pallas import tpu_sc as plsc

F32 = jnp.float32

NN = 10000
NE = 160000
D = 32
S2S_ITERS = 3

_NC = 2
_NS = 16
_NW = _NC * _NS
_CH = 128
_NCHUNKS = NE // _CH
_RPW = _NCHUNKS // _NW
_XTRA = _NCHUNKS - _RPW * _NW
_NB = 3
_RPS = NN // _NS

def _sc_mesh():
    return plsc.VectorSubcoreMesh(core_axis_name="c", subcore_axis_name="s",
                                  num_cores=_NC, num_subcores=_NS)

TE = 4000
TN = 2000


def _silu(x):
    return x * (0.5 * jnp.tanh(0.5 * x) + 0.5)


def _worker_range(wid):
    start = _RPW * wid + jnp.minimum(wid, _XTRA)
    cnt = _RPW + jnp.where(wid < _XTRA, 1, 0)
    return start, cnt


def _gather2_body(v_hbm, src_hbm, dst_hbm, os_hbm, od_hbm,
                  is0, is1, is2, id0, id1, id2,
                  rs0, rs1, rs2, rd0, rd1, rd2,
                  ss0, ss1, ss2, sd0, sd1, sd2):
    wid = lax.axis_index("s") * _NC + lax.axis_index("c")
    start_row, cnt = _worker_range(wid)
    isv = (is0, is1, is2)
    idv = (id0, id1, id2)
    rs = (rs0, rs1, rs2)
    rd = (rd0, rd1, rd2)
    ss = (ss0, ss1, ss2)
    sd = (sd0, sd1, sd2)

    def start(j, b):
        pltpu.sync_copy(src_hbm.at[start_row + j], isv[b])
        pltpu.sync_copy(dst_hbm.at[start_row + j], idv[b])
        pltpu.async_copy(v_hbm.at[isv[b]], rs[b], ss[b])
        pltpu.async_copy(v_hbm.at[idv[b]], rd[b], sd[b])

    def finish(j, b):
        pltpu.make_async_copy(v_hbm.at[isv[b]], rs[b], ss[b]).wait()
        pltpu.make_async_copy(v_hbm.at[idv[b]], rd[b], sd[b]).wait()
        off = (start_row + j) * _CH
        pltpu.sync_copy(rs[b], os_hbm.at[pl.ds(off, _CH)])
        pltpu.sync_copy(rd[b], od_hbm.at[pl.ds(off, _CH)])

    for b in range(_NB):
        start(b, b)

    def body(t, carry):
        for b in range(_NB):
            j = t * _NB + b
            finish(j, b)
            nj = j + _NB

            @pl.when(nj < cnt)
            def _():
                start(nj, b)
        return carry

    lax.fori_loop(0, _RPW // _NB, body, 0)

    @pl.when(cnt > _RPW)
    def _():
        finish(_RPW, 0)


def _seg_sum_body(m_hbm, dst_hbm, zeros_hbm, out_hbm,
                  id0, id1, id2, r0, r1, r2, shared, s0, s1, s2):
    cid = lax.axis_index("c")
    sid = lax.axis_index("s")
    wid = sid * _NC + cid
    start_row, cnt = _worker_range(wid)
    pltpu.sync_copy(zeros_hbm, shared.at[pl.ds(sid * _RPS, _RPS)])
    plsc.subcore_barrier()
    idv = (id0, id1, id2)
    rr = (r0, r1, r2)
    ss = (s0, s1, s2)

    def start(j, b):
        off = (start_row + j) * _CH
        pltpu.sync_copy(dst_hbm.at[start_row + j], idv[b])
        pltpu.async_copy(m_hbm.at[pl.ds(off, _CH)], rr[b], ss[b])

    def finish(j, b):
        off = (start_row + j) * _CH
        pltpu.make_async_copy(m_hbm.at[pl.ds(off, _CH)], rr[b], ss[b]).wait()
        pltpu.sync_copy(rr[b], shared.at[idv[b]], add=True)

    for b in range(_NB):
        start(b, b)

    def body(t, carry):
        for b in range(_NB):
            j = t * _NB + b
            finish(j, b)
            nj = j + _NB

            @pl.when(nj < cnt)
            def _():
                start(nj, b)
        return carry

    lax.fori_loop(0, _RPW // _NB, body, 0)

    @pl.when(cnt > _RPW)
    def _():
        finish(_RPW, 0)
    plsc.subcore_barrier()
    out_off = cid * NN + sid * _RPS
    pltpu.sync_copy(shared.at[pl.ds(sid * _RPS, _RPS)],
                    out_hbm.at[pl.ds(out_off, _RPS)])


def _degree_body(dst_hbm, ones_hbm, zeros_hbm, out_hbm, idxd, ones_v, shared):
    cid = lax.axis_index("c")
    sid = lax.axis_index("s")
    wid = sid * _NC + cid
    start_row, cnt = _worker_range(wid)
    pltpu.sync_copy(zeros_hbm, shared.at[pl.ds(sid * _RPS, _RPS)])
    pltpu.sync_copy(ones_hbm, ones_v)
    plsc.subcore_barrier()

    def body(j, carry):
        @pl.when(j < cnt)
        def _():
            pltpu.sync_copy(dst_hbm.at[start_row + j], idxd)
            pltpu.sync_copy(ones_v, shared.at[idxd], add=True)
        return carry

    lax.fori_loop(0, _RPW + 1, body, 0)
    plsc.subcore_barrier()
    out_off = cid * NN + sid * _RPS
    pltpu.sync_copy(shared.at[pl.ds(sid * _RPS, _RPS)],
                    out_hbm.at[pl.ds(out_off, _RPS)])


_sc_built = {}


def _sc_gather2(v, src, dst):
    if "g" not in _sc_built:
        _sc_built["g"] = pl.kernel(
            _gather2_body,
            out_type=(jax.ShapeDtypeStruct((NE, D), F32),
                      jax.ShapeDtypeStruct((NE, D), F32)),
            mesh=_sc_mesh(),
            scratch_types=([pltpu.VMEM((_CH,), jnp.int32)] * 6
                           + [pltpu.VMEM((_CH, D), F32)] * 6
                           + [pltpu.SemaphoreType.DMA] * 6),
            compiler_params=pltpu.CompilerParams(use_tc_tiling_on_sc=False))
    return _sc_built["g"](v, src, dst)


def _sc_seg_sum(m, dst, zeros_d):
    if "s" not in _sc_built:
        _sc_built["s"] = pl.kernel(
            _seg_sum_body,
            out_type=jax.ShapeDtypeStruct((_NC * NN, D), F32),
            mesh=_sc_mesh(),
            scratch_types=([pltpu.VMEM((_CH,), jnp.int32)] * 3
                           + [pltpu.VMEM((_CH, D), F32)] * 3
                           + [pltpu.VMEM_SHARED((NN, D), F32)]
                           + [pltpu.SemaphoreType.DMA] * 3),
            compiler_params=pltpu.CompilerParams(use_tc_tiling_on_sc=False))
    return _sc_built["s"](m, dst, zeros_d)


def _sc_degree(dst, ones16, zeros16):
    if "d" not in _sc_built:
        _sc_built["d"] = pl.kernel(
            _degree_body,
            out_type=jax.ShapeDtypeStruct((_NC * NN, 16), F32),
            mesh=_sc_mesh(),
            scratch_types=[pltpu.VMEM((_CH,), jnp.int32),
                           pltpu.VMEM((_CH, 16), F32),
                           pltpu.VMEM_SHARED((NN, 16), F32)],
            compiler_params=pltpu.CompilerParams(use_tc_tiling_on_sc=False))
    return _sc_built["d"](dst, ones16, zeros16)


def _wspec(w):
    return pl.BlockSpec(w.shape, lambda i: (0, 0))


def _mlp2(x, w1, b1, w2, b2, tile):
    n, din = x.shape
    dh = w1.shape[1]
    do = w2.shape[1]

    def body(x_ref, w1r, b1r, w2r, b2r, o_ref):
        h = _silu(jnp.dot(x_ref[...], w1r[...],
                          preferred_element_type=F32) + b1r[...])
        o_ref[...] = _silu(jnp.dot(h, w2r[...],
                                   preferred_element_type=F32) + b2r[...])

    return pl.pallas_call(
        body,
        grid=(n // tile,),
        in_specs=[pl.BlockSpec((tile, din), lambda i: (i, 0)),
                  _wspec(w1), _wspec(b1), _wspec(w2), _wspec(b2)],
        out_specs=pl.BlockSpec((tile, do), lambda i: (i, 0)),
        out_shape=jax.ShapeDtypeStruct((n, do), F32),
    )(x, w1, b1, w2, b2)


def _edge_conv(x, vs, vd, p1, p1b, p2, p2b, wa, wb, wc, b1, w2, b2, w3, b3,
               res_pre):
    n, din = x.shape

    def body(x_ref, vs_ref, vd_ref, p1r, p1br, p2r, p2br,
             war, wbr, wcr, b1r, w2r, b2r, w3r, b3r,
             m_o, e_o, ms_o):
        xv = x_ref[...]
        ep = _silu(jnp.dot(xv, p1r[...], preferred_element_type=F32) + p1br[...])
        ep = _silu(jnp.dot(ep, p2r[...], preferred_element_type=F32) + p2br[...])
        h = (jnp.dot(vs_ref[...], war[...], preferred_element_type=F32)
             + jnp.dot(vd_ref[...], wbr[...], preferred_element_type=F32)
             + jnp.dot(ep, wcr[...], preferred_element_type=F32) + b1r[...])
        h = _silu(h)
        h = _silu(jnp.dot(h, w2r[...], preferred_element_type=F32) + b2r[...])
        m = _silu(jnp.dot(h, w3r[...], preferred_element_type=F32) + b3r[...])
        m_o[...] = m
        e_o[...] = m + (ep if res_pre else xv)
        i = pl.program_id(0)

        @pl.when(i == 0)
        def _():
            ms_o[...] = jnp.zeros_like(ms_o)
        ms_o[...] += jnp.sum(m, axis=0, keepdims=True)

    return pl.pallas_call(
        body,
        grid=(n // TE,),
        in_specs=[pl.BlockSpec((TE, din), lambda i: (i, 0)),
                  pl.BlockSpec((TE, D), lambda i: (i, 0)),
                  pl.BlockSpec((TE, D), lambda i: (i, 0)),
                  _wspec(p1), _wspec(p1b), _wspec(p2), _wspec(p2b),
                  _wspec(wa), _wspec(wb), _wspec(wc), _wspec(b1),
                  _wspec(w2), _wspec(b2), _wspec(w3), _wspec(b3)],
        out_specs=[pl.BlockSpec((TE, D), lambda i: (i, 0)),
                   pl.BlockSpec((TE, D), lambda i: (i, 0)),
                   pl.BlockSpec((1, D), lambda i: (0, 0))],
        out_shape=[jax.ShapeDtypeStruct((n, D), F32),
                   jax.ShapeDtypeStruct((n, D), F32),
                   jax.ShapeDtypeStruct((1, D), F32)],
    )(x, vs, vd, p1, p1b, p2, p2b, wa, wb, wc, b1, w2, b2, w3, b3)


def _node_conv(v_res, vp, agg0, agg1, deg0, deg1,
               wv, wve, b1, w2, b2, w3, b3):
    def body(vr_ref, vp_ref, a0_ref, a1_ref, d0_ref, d1_ref,
             wvr, wver, b1r, w2r, b2r, w3r, b3r, v_o, vs_o):
        deg = d0_ref[...][:, 0:1] + d1_ref[...][:, 0:1]
        ve = (a0_ref[...] + a1_ref[...]) / jnp.maximum(deg, 1.0)
        h = (jnp.dot(vp_ref[...], wvr[...], preferred_element_type=F32)
             + jnp.dot(ve, wver[...], preferred_element_type=F32) + b1r[...])
        h = _silu(h)
        h = _silu(jnp.dot(h, w2r[...], preferred_element_type=F32) + b2r[...])
        vn = _silu(jnp.dot(h, w3r[...], preferred_element_type=F32) + b3r[...])
        v_o[...] = vn + vr_ref[...]
        i = pl.program_id(0)

        @pl.when(i == 0)
        def _():
            vs_o[...] = jnp.zeros_like(vs_o)
        vs_o[...] += jnp.sum(vn, axis=0, keepdims=True)

    return pl.pallas_call(
        body,
        grid=(NN // TN,),
        in_specs=[pl.BlockSpec((TN, D), lambda i: (i, 0)),
                  pl.BlockSpec((TN, D), lambda i: (i, 0)),
                  pl.BlockSpec((TN, D), lambda i: (i, 0)),
                  pl.BlockSpec((TN, D), lambda i: (i, 0)),
                  pl.BlockSpec((TN, 16), lambda i: (i, 0)),
                  pl.BlockSpec((TN, 16), lambda i: (i, 0)),
                  _wspec(wv), _wspec(wve), _wspec(b1),
                  _wspec(w2), _wspec(b2), _wspec(w3), _wspec(b3)],
        out_specs=[pl.BlockSpec((TN, D), lambda i: (i, 0)),
                   pl.BlockSpec((1, D), lambda i: (0, 0))],
        out_shape=[jax.ShapeDtypeStruct((NN, D), F32),
                   jax.ShapeDtypeStruct((1, D), F32)],
    )(v_res, vp, agg0, agg1, deg0, deg1, wv, wve, b1, w2, b2, w3, b3)


def _s2s_pass(feat, q, tile):
    n = feat.shape[0]
    grid = n // tile

    def body(f_ref, q_ref, out_ref, m_ref, z_ref, r_ref):
        i = pl.program_id(0)

        @pl.when(i == 0)
        def _():
            m_ref[...] = jnp.full_like(m_ref, -1e30)
            z_ref[...] = jnp.zeros_like(z_ref)
            r_ref[...] = jnp.zeros_like(r_ref)

        f = f_ref[...]
        s = jnp.sum(f * q_ref[...], axis=1, keepdims=True)
        m_old = m_ref[...]
        m_new = jnp.maximum(m_old, jnp.max(s))
        corr = jnp.exp(m_old - m_new)
        pexp = jnp.exp(s - m_new)
        z_new = z_ref[...] * corr + jnp.sum(pexp)
        r_new = r_ref[...] * corr + jnp.sum(pexp * f, axis=0, keepdims=True)
        m_ref[...] = m_new
        z_ref[...] = z_new
        r_ref[...] = r_new
        out_ref[...] = r_new / z_new

    out = pl.pallas_call(
        body,
        grid=(grid,),
        in_specs=[pl.BlockSpec((tile, D), lambda i: (i, 0)),
                  pl.BlockSpec((1, D), lambda i: (0, 0))],
        out_specs=[pl.BlockSpec((1, D), lambda i: (0, 0)),
                   pl.BlockSpec((1, 1), lambda i: (0, 0)),
                   pl.BlockSpec((1, 1), lambda i: (0, 0)),
                   pl.BlockSpec((1, D), lambda i: (0, 0))],
        out_shape=[jax.ShapeDtypeStruct((1, D), F32),
                   jax.ShapeDtypeStruct((1, 1), F32),
                   jax.ShapeDtypeStruct((1, 1), F32),
                   jax.ShapeDtypeStruct((1, D), F32)],
    )(feat, q)
    return out[0]


def _mlp_rows(ps, x, activate_last):
    n = len(ps)
    for i, p in enumerate(ps):
        x = x @ p["W"].T + p["b"]
        if i < n - 1 or activate_last:
            x = _silu(x)
    return x


def _lstm(x, hs, cs, layers):
    new_h, new_c = [], []
    for l, p in enumerate(layers):
        g = x @ p["W_ih"].T + p["b_ih"] + hs[l] @ p["W_hh"].T + p["b_hh"]
        i, f, gg, o = jnp.split(g, 4, axis=-1)
        c = jax.nn.sigmoid(f) * cs[l] + jax.nn.sigmoid(i) * jnp.tanh(gg)
        h = jax.nn.sigmoid(o) * jnp.tanh(c)
        new_h.append(h)
        new_c.append(c)
        x = h
    return x, new_h, new_c


def _set2set(feat, layers, tile):
    d = feat.shape[-1]
    hs = [jnp.zeros((1, d), F32) for _ in layers]
    cs = [jnp.zeros((1, d), F32) for _ in layers]
    q_star = jnp.zeros((1, 2 * d), F32)
    for _ in range(S2S_ITERS):
        q, hs, cs = _lstm(q_star, hs, cs, layers)
        readout = _s2s_pass(feat, q, tile)
        q_star = jnp.concatenate([q, readout], axis=-1)
    return q_star


def _t(p):
    return p["W"].T


def _b(p):
    return p["b"][None, :]


def kernel(edge_feat, node_feat, graph_attr, edge_index, params):
    p = params
    src2 = edge_index[0].reshape(_NCHUNKS, _CH)
    dst2 = edge_index[1].reshape(_NCHUNKS, _CH)
    zeros_d = jnp.zeros((_RPS, D), F32)
    zeros16 = jnp.zeros((_RPS, 16), F32)
    ones16 = jnp.ones((_CH, 16), F32)

    degp = _sc_degree(dst2, ones16, zeros16)
    deg0 = degp[:NN]
    deg1 = degp[NN:]

    en = p["node_encoder"]
    v = _mlp2(node_feat, _t(en[0]), _b(en[0]), _t(en[1]), _b(en[1]), TN)
    u = _mlp_rows(p["attr_encoder"], graph_attr, True)
    e = None

    for bi, bp in enumerate(p["blocks"]):
        u0 = u
        if bi > 0:
            pn = bp["pre_node"]
            vp = _mlp2(v, _t(pn[0]), _b(pn[0]), _t(pn[1]), _b(pn[1]), TN)
            u_in = _mlp_rows(bp["pre_attr"], u, True)
            ex = e
            pre = bp["pre_edge"]
        else:
            vp = v
            u_in = u
            ex = edge_feat
            pre = p["edge_encoder"]

        vs, vd = _sc_gather2(vp, src2, dst2)

        ce = bp["conv_edge"]
        w1 = ce[0]["W"]
        b1e = ce[0]["b"][None, :] + u_in @ w1[:, 3 * D:4 * D].T
        m, e_new, msum = _edge_conv(
            ex, vs, vd,
            _t(pre[0]), _b(pre[0]), _t(pre[1]), _b(pre[1]),
            w1[:, 0:D].T, w1[:, D:2 * D].T, w1[:, 2 * D:3 * D].T, b1e,
            _t(ce[1]), _b(ce[1]), _t(ce[2]), _b(ce[2]),
            res_pre=(bi == 0))

        aggp = _sc_seg_sum(m, dst2, zeros_d)

        cn = bp["conv_node"]
        w1n = cn[0]["W"]
        b1n = cn[0]["b"][None, :] + u_in @ w1n[:, 2 * D:3 * D].T
        v_new, vsum = _node_conv(
            v, vp, aggp[:NN], aggp[NN:], deg0, deg1,
            w1n[:, 0:D].T, w1n[:, D:2 * D].T, b1n,
            _t(cn[1]), _b(cn[1]), _t(cn[2]), _b(cn[2]))

        mean_v = vsum / NN
        mean_e = msum / NE
        u = _mlp_rows(bp["conv_attr"],
                      jnp.concatenate([u_in, mean_v, mean_e], axis=-1),
                      True) + u0
        v = v_new
        e = e_new

    node_vec = _set2set(v, p["node_s2s_lstm"], TN)
    edge_vec = _set2set(e, p["edge_s2s_lstm"], TE)
    vec = jnp.concatenate([node_vec[0], edge_vec[0], u[0]], axis=-1)
    return _mlp_rows(p["output_proj"], vec, False)


# speedup vs baseline: 2.3625x; 1.0239x over previous
import functools

import jax
import jax.numpy as jnp
from jax import lax
from jax.experimental import pallas as pl
from jax.experimental.pallas import tpu as pltpu
from jax.experimental.pallas import tpu_sc as plsc

F32 = jnp.float32

NN = 10000
NE = 160000
D = 32
S2S_ITERS = 3

_NC = 2
_NS = 16
_NW = _NC * _NS
_CH = 128
_NCHUNKS = NE // _CH
_RPW = _NCHUNKS // _NW
_XTRA = _NCHUNKS - _RPW * _NW
_NB = 3
_RPS = NN // _NS

def _sc_mesh():
    return plsc.VectorSubcoreMesh(core_axis_name="c", subcore_axis_name="s",
                                  num_cores=_NC, num_subcores=_NS)

TE = 8000
TN = 2000


def _silu(x):
    return x * (0.5 * jnp.tanh(0.5 * x) + 0.5)


def _worker_range(wid):
    start = _RPW * wid + jnp.minimum(wid, _XTRA)
    cnt = _RPW + jnp.where(wid < _XTRA, 1, 0)
    return start, cnt


def _gather2_body(v_hbm, src_hbm, dst_hbm, os_hbm, od_hbm,
                  is0, is1, is2, id0, id1, id2,
                  rs0, rs1, rs2, rd0, rd1, rd2,
                  ss0, ss1, ss2, sd0, sd1, sd2):
    wid = lax.axis_index("s") * _NC + lax.axis_index("c")
    start_row, cnt = _worker_range(wid)
    isv = (is0, is1, is2)
    idv = (id0, id1, id2)
    rs = (rs0, rs1, rs2)
    rd = (rd0, rd1, rd2)
    ss = (ss0, ss1, ss2)
    sd = (sd0, sd1, sd2)

    def start(j, b):
        pltpu.sync_copy(src_hbm.at[start_row + j], isv[b])
        pltpu.sync_copy(dst_hbm.at[start_row + j], idv[b])
        pltpu.async_copy(v_hbm.at[isv[b]], rs[b], ss[b])
        pltpu.async_copy(v_hbm.at[idv[b]], rd[b], sd[b])

    def finish(j, b):
        pltpu.make_async_copy(v_hbm.at[isv[b]], rs[b], ss[b]).wait()
        pltpu.make_async_copy(v_hbm.at[idv[b]], rd[b], sd[b]).wait()
        off = (start_row + j) * _CH
        pltpu.sync_copy(rs[b], os_hbm.at[pl.ds(off, _CH)])
        pltpu.sync_copy(rd[b], od_hbm.at[pl.ds(off, _CH)])

    for b in range(_NB):
        start(b, b)

    def body(t, carry):
        for b in range(_NB):
            j = t * _NB + b
            finish(j, b)
            nj = j + _NB

            @pl.when(nj < cnt)
            def _():
                start(nj, b)
        return carry

    lax.fori_loop(0, _RPW // _NB, body, 0)

    @pl.when(cnt > _RPW)
    def _():
        finish(_RPW, 0)


def _seg_sum_body(m_hbm, dst_hbm, zeros_hbm, out_hbm,
                  id0, id1, id2, r0, r1, r2, shared, s0, s1, s2):
    cid = lax.axis_index("c")
    sid = lax.axis_index("s")
    wid = sid * _NC + cid
    start_row, cnt = _worker_range(wid)
    pltpu.sync_copy(zeros_hbm, shared.at[pl.ds(sid * _RPS, _RPS)])
    plsc.subcore_barrier()
    idv = (id0, id1, id2)
    rr = (r0, r1, r2)
    ss = (s0, s1, s2)

    def start(j, b):
        off = (start_row + j) * _CH
        pltpu.sync_copy(dst_hbm.at[start_row + j], idv[b])
        pltpu.async_copy(m_hbm.at[pl.ds(off, _CH)], rr[b], ss[b])

    def finish(j, b):
        off = (start_row + j) * _CH
        pltpu.make_async_copy(m_hbm.at[pl.ds(off, _CH)], rr[b], ss[b]).wait()
        pltpu.sync_copy(rr[b], shared.at[idv[b]], add=True)

    for b in range(_NB):
        start(b, b)

    def body(t, carry):
        for b in range(_NB):
            j = t * _NB + b
            finish(j, b)
            nj = j + _NB

            @pl.when(nj < cnt)
            def _():
                start(nj, b)
        return carry

    lax.fori_loop(0, _RPW // _NB, body, 0)

    @pl.when(cnt > _RPW)
    def _():
        finish(_RPW, 0)
    plsc.subcore_barrier()
    out_off = cid * NN + sid * _RPS
    pltpu.sync_copy(shared.at[pl.ds(sid * _RPS, _RPS)],
                    out_hbm.at[pl.ds(out_off, _RPS)])


def _degree_body(dst_hbm, ones_hbm, zeros_hbm, out_hbm, idxd, ones_v, shared):
    cid = lax.axis_index("c")
    sid = lax.axis_index("s")
    wid = sid * _NC + cid
    start_row, cnt = _worker_range(wid)
    pltpu.sync_copy(zeros_hbm, shared.at[pl.ds(sid * _RPS, _RPS)])
    pltpu.sync_copy(ones_hbm, ones_v)
    plsc.subcore_barrier()

    def body(j, carry):
        @pl.when(j < cnt)
        def _():
            pltpu.sync_copy(dst_hbm.at[start_row + j], idxd)
            pltpu.sync_copy(ones_v, shared.at[idxd], add=True)
        return carry

    lax.fori_loop(0, _RPW + 1, body, 0)
    plsc.subcore_barrier()
    out_off = cid * NN + sid * _RPS
    pltpu.sync_copy(shared.at[pl.ds(sid * _RPS, _RPS)],
                    out_hbm.at[pl.ds(out_off, _RPS)])


_sc_built = {}


def _sc_gather2(v, src, dst):
    if "g" not in _sc_built:
        _sc_built["g"] = pl.kernel(
            _gather2_body,
            out_type=(jax.ShapeDtypeStruct((NE, D), F32),
                      jax.ShapeDtypeStruct((NE, D), F32)),
            mesh=_sc_mesh(),
            scratch_types=([pltpu.VMEM((_CH,), jnp.int32)] * 6
                           + [pltpu.VMEM((_CH, D), F32)] * 6
                           + [pltpu.SemaphoreType.DMA] * 6),
            compiler_params=pltpu.CompilerParams(use_tc_tiling_on_sc=False))
    return _sc_built["g"](v, src, dst)


def _sc_seg_sum(m, dst, zeros_d):
    if "s" not in _sc_built:
        _sc_built["s"] = pl.kernel(
            _seg_sum_body,
            out_type=jax.ShapeDtypeStruct((_NC * NN, D), F32),
            mesh=_sc_mesh(),
            scratch_types=([pltpu.VMEM((_CH,), jnp.int32)] * 3
                           + [pltpu.VMEM((_CH, D), F32)] * 3
                           + [pltpu.VMEM_SHARED((NN, D), F32)]
                           + [pltpu.SemaphoreType.DMA] * 3),
            compiler_params=pltpu.CompilerParams(use_tc_tiling_on_sc=False))
    return _sc_built["s"](m, dst, zeros_d)


def _sc_degree(dst, ones16, zeros16):
    if "d" not in _sc_built:
        _sc_built["d"] = pl.kernel(
            _degree_body,
            out_type=jax.ShapeDtypeStruct((_NC * NN, 16), F32),
            mesh=_sc_mesh(),
            scratch_types=[pltpu.VMEM((_CH,), jnp.int32),
                           pltpu.VMEM((_CH, 16), F32),
                           pltpu.VMEM_SHARED((NN, 16), F32)],
            compiler_params=pltpu.CompilerParams(use_tc_tiling_on_sc=False))
    return _sc_built["d"](dst, ones16, zeros16)


def _wspec(w):
    return pl.BlockSpec(w.shape, lambda i: (0, 0))


def _mlp2(x, w1, b1, w2, b2, tile):
    n, din = x.shape
    dh = w1.shape[1]
    do = w2.shape[1]

    def body(x_ref, w1r, b1r, w2r, b2r, o_ref):
        h = _silu(jnp.dot(x_ref[...], w1r[...],
                          preferred_element_type=F32) + b1r[...])
        o_ref[...] = _silu(jnp.dot(h, w2r[...],
                                   preferred_element_type=F32) + b2r[...])

    return pl.pallas_call(
        body,
        grid=(n // tile,),
        in_specs=[pl.BlockSpec((tile, din), lambda i: (i, 0)),
                  _wspec(w1), _wspec(b1), _wspec(w2), _wspec(b2)],
        out_specs=pl.BlockSpec((tile, do), lambda i: (i, 0)),
        out_shape=jax.ShapeDtypeStruct((n, do), F32),
    )(x, w1, b1, w2, b2)


def _edge_conv(x, vs, vd, p1, p1b, p2, p2b, wa, wb, wc, b1, w2, b2, w3, b3,
               res_pre):
    n, din = x.shape

    def body(x_ref, vs_ref, vd_ref, p1r, p1br, p2r, p2br,
             war, wbr, wcr, b1r, w2r, b2r, w3r, b3r,
             m_o, e_o, ms_o):
        xv = x_ref[...]
        ep = _silu(jnp.dot(xv, p1r[...], preferred_element_type=F32) + p1br[...])
        ep = _silu(jnp.dot(ep, p2r[...], preferred_element_type=F32) + p2br[...])
        h = (jnp.dot(vs_ref[...], war[...], preferred_element_type=F32)
             + jnp.dot(vd_ref[...], wbr[...], preferred_element_type=F32)
             + jnp.dot(ep, wcr[...], preferred_element_type=F32) + b1r[...])
        h = _silu(h)
        h = _silu(jnp.dot(h, w2r[...], preferred_element_type=F32) + b2r[...])
        m = _silu(jnp.dot(h, w3r[...], preferred_element_type=F32) + b3r[...])
        m_o[...] = m
        e_o[...] = m + (ep if res_pre else xv)
        i = pl.program_id(0)

        @pl.when(i == 0)
        def _():
            ms_o[...] = jnp.zeros_like(ms_o)
        ms_o[...] += jnp.sum(m, axis=0, keepdims=True)

    return pl.pallas_call(
        body,
        grid=(n // TE,),
        in_specs=[pl.BlockSpec((TE, din), lambda i: (i, 0)),
                  pl.BlockSpec((TE, D), lambda i: (i, 0)),
                  pl.BlockSpec((TE, D), lambda i: (i, 0)),
                  _wspec(p1), _wspec(p1b), _wspec(p2), _wspec(p2b),
                  _wspec(wa), _wspec(wb), _wspec(wc), _wspec(b1),
                  _wspec(w2), _wspec(b2), _wspec(w3), _wspec(b3)],
        out_specs=[pl.BlockSpec((TE, D), lambda i: (i, 0)),
                   pl.BlockSpec((TE, D), lambda i: (i, 0)),
                   pl.BlockSpec((1, D), lambda i: (0, 0))],
        out_shape=[jax.ShapeDtypeStruct((n, D), F32),
                   jax.ShapeDtypeStruct((n, D), F32),
                   jax.ShapeDtypeStruct((1, D), F32)],
    )(x, vs, vd, p1, p1b, p2, p2b, wa, wb, wc, b1, w2, b2, w3, b3)


def _node_conv(v_res, vp, agg0, agg1, deg0, deg1,
               wv, wve, b1, w2, b2, w3, b3):
    def body(vr_ref, vp_ref, a0_ref, a1_ref, d0_ref, d1_ref,
             wvr, wver, b1r, w2r, b2r, w3r, b3r, v_o, vs_o):
        deg = d0_ref[...][:, 0:1] + d1_ref[...][:, 0:1]
        ve = (a0_ref[...] + a1_ref[...]) / jnp.maximum(deg, 1.0)
        h = (jnp.dot(vp_ref[...], wvr[...], preferred_element_type=F32)
             + jnp.dot(ve, wver[...], preferred_element_type=F32) + b1r[...])
        h = _silu(h)
        h = _silu(jnp.dot(h, w2r[...], preferred_element_type=F32) + b2r[...])
        vn = _silu(jnp.dot(h, w3r[...], preferred_element_type=F32) + b3r[...])
        v_o[...] = vn + vr_ref[...]
        i = pl.program_id(0)

        @pl.when(i == 0)
        def _():
            vs_o[...] = jnp.zeros_like(vs_o)
        vs_o[...] += jnp.sum(vn, axis=0, keepdims=True)

    return pl.pallas_call(
        body,
        grid=(NN // TN,),
        in_specs=[pl.BlockSpec((TN, D), lambda i: (i, 0)),
                  pl.BlockSpec((TN, D), lambda i: (i, 0)),
                  pl.BlockSpec((TN, D), lambda i: (i, 0)),
                  pl.BlockSpec((TN, D), lambda i: (i, 0)),
                  pl.BlockSpec((TN, 16), lambda i: (i, 0)),
                  pl.BlockSpec((TN, 16), lambda i: (i, 0)),
                  _wspec(wv), _wspec(wve), _wspec(b1),
                  _wspec(w2), _wspec(b2), _wspec(w3), _wspec(b3)],
        out_specs=[pl.BlockSpec((TN, D), lambda i: (i, 0)),
                   pl.BlockSpec((1, D), lambda i: (0, 0))],
        out_shape=[jax.ShapeDtypeStruct((NN, D), F32),
                   jax.ShapeDtypeStruct((1, D), F32)],
    )(v_res, vp, agg0, agg1, deg0, deg1, wv, wve, b1, w2, b2, w3, b3)


def _s2s_pass(feat, q, tile):
    n = feat.shape[0]
    grid = n // tile

    def body(f_ref, q_ref, out_ref, m_ref, z_ref, r_ref):
        i = pl.program_id(0)

        @pl.when(i == 0)
        def _():
            m_ref[...] = jnp.full_like(m_ref, -1e30)
            z_ref[...] = jnp.zeros_like(z_ref)
            r_ref[...] = jnp.zeros_like(r_ref)

        f = f_ref[...]
        s = jnp.sum(f * q_ref[...], axis=1, keepdims=True)
        m_old = m_ref[...]
        m_new = jnp.maximum(m_old, jnp.max(s))
        corr = jnp.exp(m_old - m_new)
        pexp = jnp.exp(s - m_new)
        z_new = z_ref[...] * corr + jnp.sum(pexp)
        r_new = r_ref[...] * corr + jnp.sum(pexp * f, axis=0, keepdims=True)
        m_ref[...] = m_new
        z_ref[...] = z_new
        r_ref[...] = r_new
        out_ref[...] = r_new / z_new

    out = pl.pallas_call(
        body,
        grid=(grid,),
        in_specs=[pl.BlockSpec((tile, D), lambda i: (i, 0)),
                  pl.BlockSpec((1, D), lambda i: (0, 0))],
        out_specs=[pl.BlockSpec((1, D), lambda i: (0, 0)),
                   pl.BlockSpec((1, 1), lambda i: (0, 0)),
                   pl.BlockSpec((1, 1), lambda i: (0, 0)),
                   pl.BlockSpec((1, D), lambda i: (0, 0))],
        out_shape=[jax.ShapeDtypeStruct((1, D), F32),
                   jax.ShapeDtypeStruct((1, 1), F32),
                   jax.ShapeDtypeStruct((1, 1), F32),
                   jax.ShapeDtypeStruct((1, D), F32)],
    )(feat, q)
    return out[0]


def _mlp_rows(ps, x, activate_last):
    n = len(ps)
    for i, p in enumerate(ps):
        x = x @ p["W"].T + p["b"]
        if i < n - 1 or activate_last:
            x = _silu(x)
    return x


def _lstm(x, hs, cs, layers):
    new_h, new_c = [], []
    for l, p in enumerate(layers):
        g = x @ p["W_ih"].T + p["b_ih"] + hs[l] @ p["W_hh"].T + p["b_hh"]
        i, f, gg, o = jnp.split(g, 4, axis=-1)
        c = jax.nn.sigmoid(f) * cs[l] + jax.nn.sigmoid(i) * jnp.tanh(gg)
        h = jax.nn.sigmoid(o) * jnp.tanh(c)
        new_h.append(h)
        new_c.append(c)
        x = h
    return x, new_h, new_c


def _set2set(feat, layers, tile):
    d = feat.shape[-1]
    hs = [jnp.zeros((1, d), F32) for _ in layers]
    cs = [jnp.zeros((1, d), F32) for _ in layers]
    q_star = jnp.zeros((1, 2 * d), F32)
    for _ in range(S2S_ITERS):
        q, hs, cs = _lstm(q_star, hs, cs, layers)
        readout = _s2s_pass(feat, q, tile)
        q_star = jnp.concatenate([q, readout], axis=-1)
    return q_star


def _t(p):
    return p["W"].T


def _b(p):
    return p["b"][None, :]


def kernel(edge_feat, node_feat, graph_attr, edge_index, params):
    p = params
    src2 = edge_index[0].reshape(_NCHUNKS, _CH)
    dst2 = edge_index[1].reshape(_NCHUNKS, _CH)
    zeros_d = jnp.zeros((_RPS, D), F32)
    zeros16 = jnp.zeros((_RPS, 16), F32)
    ones16 = jnp.ones((_CH, 16), F32)

    degp = _sc_degree(dst2, ones16, zeros16)
    deg0 = degp[:NN]
    deg1 = degp[NN:]

    en = p["node_encoder"]
    v = _mlp2(node_feat, _t(en[0]), _b(en[0]), _t(en[1]), _b(en[1]), TN)
    u = _mlp_rows(p["attr_encoder"], graph_attr, True)
    e = None

    for bi, bp in enumerate(p["blocks"]):
        u0 = u
        if bi > 0:
            pn = bp["pre_node"]
            vp = _mlp2(v, _t(pn[0]), _b(pn[0]), _t(pn[1]), _b(pn[1]), TN)
            u_in = _mlp_rows(bp["pre_attr"], u, True)
            ex = e
            pre = bp["pre_edge"]
        else:
            vp = v
            u_in = u
            ex = edge_feat
            pre = p["edge_encoder"]

        vs, vd = _sc_gather2(vp, src2, dst2)

        ce = bp["conv_edge"]
        w1 = ce[0]["W"]
        b1e = ce[0]["b"][None, :] + u_in @ w1[:, 3 * D:4 * D].T
        m, e_new, msum = _edge_conv(
            ex, vs, vd,
            _t(pre[0]), _b(pre[0]), _t(pre[1]), _b(pre[1]),
            w1[:, 0:D].T, w1[:, D:2 * D].T, w1[:, 2 * D:3 * D].T, b1e,
            _t(ce[1]), _b(ce[1]), _t(ce[2]), _b(ce[2]),
            res_pre=(bi == 0))

        aggp = _sc_seg_sum(m, dst2, zeros_d)

        cn = bp["conv_node"]
        w1n = cn[0]["W"]
        b1n = cn[0]["b"][None, :] + u_in @ w1n[:, 2 * D:3 * D].T
        v_new, vsum = _node_conv(
            v, vp, aggp[:NN], aggp[NN:], deg0, deg1,
            w1n[:, 0:D].T, w1n[:, D:2 * D].T, b1n,
            _t(cn[1]), _b(cn[1]), _t(cn[2]), _b(cn[2]))

        mean_v = vsum / NN
        mean_e = msum / NE
        u = _mlp_rows(bp["conv_attr"],
                      jnp.concatenate([u_in, mean_v, mean_e], axis=-1),
                      True) + u0
        v = v_new
        e = e_new

    node_vec = _set2set(v, p["node_s2s_lstm"], TN)
    edge_vec = _set2set(e, p["edge_s2s_lstm"], TE)
    vec = jnp.concatenate([node_vec[0], edge_vec[0], u[0]], axis=-1)
    return _mlp_rows(p["output_proj"], vec, False)
